# baseline (device time: 20546 ns/iter reference)
import os

import jax
import jax.numpy as jnp
from jax import lax
from jax.experimental import pallas as pl
from jax.experimental.pallas import tpu as pltpu

_ABLATE = os.environ.get("ABLATE", "")

N_DEV = 32
ROWS = 8

_SEND_ORDER = [14, 18, 10, 22, 13, 19, 11, 21, 12, 20, 6, 26, 5, 15, 17,
               27, 2, 30, 3, 9, 23, 29, 4, 28, 7, 25, 16, 8, 24, 1, 31]


def kernel(x, Wg, Wu, Wd):
    m, k = x.shape
    _, h_per = Wg.shape
    _, n = Wd.shape

    if _ABLATE in ("min", "min4"):
        def min_body(x_ref, *rest):
            out_ref = rest[-1]
            my_pos = lax.axis_index("i")
            barrier_sem = pltpu.get_barrier_semaphore()
            for j in range(N_DEV):
                @pl.when(j != my_pos)
                def _(j=j):
                    pl.semaphore_signal(
                        barrier_sem, inc=1,
                        device_id=(j,), device_id_type=pl.DeviceIdType.MESH,
                    )
            out_ref[...] = x_ref[...].astype(jnp.bfloat16)
            pl.semaphore_wait(barrier_sem, N_DEV - 1)
            for j in range(N_DEV):
                @pl.when(j != my_pos)
                def _(j=j):
                    pl.semaphore_signal(
                        barrier_sem, inc=1,
                        device_id=(j,), device_id_type=pl.DeviceIdType.MESH,
                    )
        n_ops = 4 if _ABLATE == "min4" else 1
        args = (x, Wg, Wu, Wd)[:n_ops]
        return pl.pallas_call(
            min_body,
            out_shape=jax.ShapeDtypeStruct((m, n), jnp.bfloat16),
            in_specs=[pl.BlockSpec(memory_space=pltpu.VMEM)] * n_ops,
            out_specs=pl.BlockSpec(memory_space=pltpu.VMEM),
            compiler_params=pltpu.CompilerParams(collective_id=0),
        )(*args)

    def body(x_ref, wg_hbm, wu_hbm, wd_hbm, out_ref,
             wg_ref, wu_ref, wd_ref, acc16_ref, comm_ref, red16_ref,
             w_sems, p1_send, p1_recv, p2_send, p2_recv):
        my_pos = lax.axis_index("i")

        w_copies = []
        for i, (src, dst) in enumerate(
            [(wg_hbm, wg_ref), (wu_hbm, wu_ref), (wd_hbm, wd_ref)]
        ):
            cp = pltpu.make_async_copy(src, dst, w_sems.at[i])
            cp.start()
            w_copies.append(cp)

        barrier_sem = pltpu.get_barrier_semaphore()
        for j in range(N_DEV):
            @pl.when(j != my_pos)
            def _(j=j):
                pl.semaphore_signal(
                    barrier_sem, inc=1,
                    device_id=(j,), device_id_type=pl.DeviceIdType.MESH,
                )

        if _ABLATE == "nocompute":
            for cp in w_copies:
                cp.wait()
            acc16_ref[...] = x_ref[...].astype(jnp.bfloat16)
        else:
            xb = x_ref[...].astype(jnp.bfloat16)
            w_copies[0].wait()
            gate = jnp.dot(xb, wg_ref[...].astype(jnp.bfloat16),
                           preferred_element_type=jnp.float32)
            w_copies[1].wait()
            up = jnp.dot(xb, wu_ref[...].astype(jnp.bfloat16),
                         preferred_element_type=jnp.float32)
            hidden = (gate * (up * jax.nn.sigmoid(up))).astype(jnp.bfloat16)
            w_copies[2].wait()
            acc16_ref[...] = jnp.dot(
                hidden, wd_ref[...].astype(jnp.bfloat16),
                preferred_element_type=jnp.float32,
            ).astype(jnp.bfloat16)

        pl.semaphore_wait(barrier_sem, N_DEV - 1)

        my_idx = 0 if _ABLATE == "staticidx" else my_pos

        p1_rdmas = []
        _p1_offsets = [] if _ABLATE in ("nop1", "nocomm") else _SEND_ORDER
        for off in _p1_offsets:
            j = (my_pos + off) % N_DEV
            rdma = pltpu.make_async_remote_copy(
                src_ref=acc16_ref.at[pl.ds(j * ROWS, ROWS), :],
                dst_ref=comm_ref.at[my_pos],
                send_sem=p1_send.at[off],
                recv_sem=p1_recv.at[my_pos],
                device_id=(j,),
                device_id_type=pl.DeviceIdType.MESH,
            )
            rdma.start()
            p1_rdmas.append(rdma)

        comm_ref[my_idx] = acc16_ref[pl.ds(my_idx * ROWS, ROWS), :]

        for s in ([] if _ABLATE in ("nop1", "nocomm") else range(N_DEV)):
            recv = pltpu.make_async_remote_copy(
                src_ref=comm_ref.at[s],
                dst_ref=comm_ref.at[s],
                send_sem=p1_send.at[0],
                recv_sem=p1_recv.at[s],
                device_id=(s,),
                device_id_type=pl.DeviceIdType.MESH,
            )

            @pl.when(s != my_pos)
            def _(recv=recv):
                recv.wait_recv()

        if _ABLATE == "nosum":
            red16_ref[...] = comm_ref[0]
        else:
            red = jnp.sum(comm_ref[...].astype(jnp.float32), axis=0)
            red16_ref[...] = red.astype(jnp.bfloat16)

        out_ref[pl.ds(my_idx * ROWS, ROWS), :] = red16_ref[...]
        p2_rdmas = []
        _p2_offsets = [] if _ABLATE in ("nop2", "nocomm") else _SEND_ORDER
        for off in _p2_offsets:
            j = (my_pos + off) % N_DEV
            rdma = pltpu.make_async_remote_copy(
                src_ref=red16_ref,
                dst_ref=out_ref.at[pl.ds(my_pos * ROWS, ROWS), :],
                send_sem=p2_send.at[off],
                recv_sem=p2_recv.at[my_pos],
                device_id=(j,),
                device_id_type=pl.DeviceIdType.MESH,
            )
            rdma.start()
            p2_rdmas.append(rdma)

        for s in ([] if _ABLATE in ("nop2", "nocomm") else range(N_DEV)):
            recv = pltpu.make_async_remote_copy(
                src_ref=red16_ref,
                dst_ref=out_ref.at[pl.ds(s * ROWS, ROWS), :],
                send_sem=p2_send.at[0],
                recv_sem=p2_recv.at[s],
                device_id=(s,),
                device_id_type=pl.DeviceIdType.MESH,
            )

            @pl.when(s != my_pos)
            def _(recv=recv):
                recv.wait_recv()

        for rdma in p1_rdmas:
            rdma.wait_send()
        for rdma in p2_rdmas:
            rdma.wait_send()

        for j in range(N_DEV):
            @pl.when(j != my_pos)
            def _(j=j):
                pl.semaphore_signal(
                    barrier_sem, inc=1,
                    device_id=(j,), device_id_type=pl.DeviceIdType.MESH,
                )

    return pl.pallas_call(
        body,
        out_shape=jax.ShapeDtypeStruct((m, n), jnp.bfloat16),
        in_specs=[pl.BlockSpec(memory_space=pltpu.VMEM)]
        + [pl.BlockSpec(memory_space=pltpu.MemorySpace.HBM)] * 3,
        out_specs=pl.BlockSpec(memory_space=pltpu.VMEM),
        scratch_shapes=[
            pltpu.VMEM((k, h_per), jnp.float32),
            pltpu.VMEM((k, h_per), jnp.float32),
            pltpu.VMEM((h_per, n), jnp.float32),
            pltpu.VMEM((m, n), jnp.bfloat16),
            pltpu.VMEM((N_DEV, ROWS, n), jnp.bfloat16),
            pltpu.VMEM((ROWS, n), jnp.bfloat16),
            pltpu.SemaphoreType.DMA((3,)),
            pltpu.SemaphoreType.DMA((N_DEV,)),
            pltpu.SemaphoreType.DMA((N_DEV,)),
            pltpu.SemaphoreType.DMA((N_DEV,)),
            pltpu.SemaphoreType.DMA((N_DEV,)),
        ],
        compiler_params=pltpu.CompilerParams(collective_id=0),
    )(x, Wg, Wu, Wd)


# device time: 18973 ns/iter; 1.0829x vs baseline; 1.0829x over previous
import os

import jax
import jax.numpy as jnp
from jax import lax
from jax.experimental import pallas as pl
from jax.experimental.pallas import tpu as pltpu

_ABLATE = os.environ.get("ABLATE", "")

N_DEV = 32
ROWS = 8

_SEND_ORDER = [14, 18, 10, 22, 13, 19, 11, 21, 12, 20, 6, 26, 5, 15, 17,
               27, 2, 30, 3, 9, 23, 29, 4, 28, 7, 25, 16, 8, 24, 1, 31]


def kernel(x, Wg, Wu, Wd):
    m, k = x.shape
    _, h_per = Wg.shape
    _, n = Wd.shape

    if _ABLATE in ("min", "min4"):
        def min_body(x_ref, *rest):
            out_ref = rest[-1]
            my_pos = lax.axis_index("i")
            barrier_sem = pltpu.get_barrier_semaphore()
            for j in range(N_DEV):
                @pl.when(j != my_pos)
                def _(j=j):
                    pl.semaphore_signal(
                        barrier_sem, inc=1,
                        device_id=(j,), device_id_type=pl.DeviceIdType.MESH,
                    )
            out_ref[...] = x_ref[...].astype(jnp.bfloat16)
            pl.semaphore_wait(barrier_sem, N_DEV - 1)
            for j in range(N_DEV):
                @pl.when(j != my_pos)
                def _(j=j):
                    pl.semaphore_signal(
                        barrier_sem, inc=1,
                        device_id=(j,), device_id_type=pl.DeviceIdType.MESH,
                    )
        n_ops = 4 if _ABLATE == "min4" else 1
        args = (x, Wg, Wu, Wd)[:n_ops]
        return pl.pallas_call(
            min_body,
            out_shape=jax.ShapeDtypeStruct((m, n), jnp.bfloat16),
            in_specs=[pl.BlockSpec(memory_space=pltpu.VMEM)] * n_ops,
            out_specs=pl.BlockSpec(memory_space=pltpu.VMEM),
            compiler_params=pltpu.CompilerParams(collective_id=0),
        )(*args)

    def body(x_ref, wg_ref, wu_ref, wd_ref, out_ref,
             acc16_ref, comm_ref, red16_ref,
             p1_send, p1_recv, p2_send, p2_recv):
        my_pos = lax.axis_index("i")

        barrier_sem = pltpu.get_barrier_semaphore()
        for j in range(N_DEV):
            @pl.when(j != my_pos)
            def _(j=j):
                pl.semaphore_signal(
                    barrier_sem, inc=1,
                    device_id=(j,), device_id_type=pl.DeviceIdType.MESH,
                )

        if _ABLATE == "nocompute":
            acc16_ref[...] = x_ref[...]
        else:
            gate = jnp.dot(x_ref[...], wg_ref[...],
                           preferred_element_type=jnp.float32)
            up = jnp.dot(x_ref[...], wu_ref[...],
                         preferred_element_type=jnp.float32)
            hidden = (gate * (up * jax.nn.sigmoid(up))).astype(jnp.bfloat16)
            acc16_ref[...] = jnp.dot(
                hidden, wd_ref[...],
                preferred_element_type=jnp.float32,
            ).astype(jnp.bfloat16)

        pl.semaphore_wait(barrier_sem, N_DEV - 1)

        my_idx = 0 if _ABLATE == "staticidx" else my_pos

        p1_rdmas = []
        _p1_offsets = [] if _ABLATE in ("nop1", "nocomm") else _SEND_ORDER
        for off in _p1_offsets:
            j = (my_pos + off) % N_DEV
            rdma = pltpu.make_async_remote_copy(
                src_ref=acc16_ref.at[pl.ds(j * ROWS, ROWS), :],
                dst_ref=comm_ref.at[my_pos],
                send_sem=p1_send.at[off],
                recv_sem=p1_recv.at[my_pos],
                device_id=(j,),
                device_id_type=pl.DeviceIdType.MESH,
            )
            rdma.start()
            p1_rdmas.append(rdma)

        comm_ref[my_idx] = acc16_ref[pl.ds(my_idx * ROWS, ROWS), :]

        for s in ([] if _ABLATE in ("nop1", "nocomm") else range(N_DEV)):
            recv = pltpu.make_async_remote_copy(
                src_ref=comm_ref.at[s],
                dst_ref=comm_ref.at[s],
                send_sem=p1_send.at[0],
                recv_sem=p1_recv.at[s],
                device_id=(s,),
                device_id_type=pl.DeviceIdType.MESH,
            )

            @pl.when(s != my_pos)
            def _(recv=recv):
                recv.wait_recv()

        if _ABLATE == "nosum":
            red16_ref[...] = comm_ref[0]
        else:
            red = jnp.sum(comm_ref[...].astype(jnp.float32), axis=0)
            red16_ref[...] = red.astype(jnp.bfloat16)

        out_ref[pl.ds(my_idx * ROWS, ROWS), :] = red16_ref[...]
        p2_rdmas = []
        _p2_offsets = [] if _ABLATE in ("nop2", "nocomm") else _SEND_ORDER
        for off in _p2_offsets:
            j = (my_pos + off) % N_DEV
            rdma = pltpu.make_async_remote_copy(
                src_ref=red16_ref,
                dst_ref=out_ref.at[pl.ds(my_pos * ROWS, ROWS), :],
                send_sem=p2_send.at[off],
                recv_sem=p2_recv.at[my_pos],
                device_id=(j,),
                device_id_type=pl.DeviceIdType.MESH,
            )
            rdma.start()
            p2_rdmas.append(rdma)

        for s in ([] if _ABLATE in ("nop2", "nocomm") else range(N_DEV)):
            recv = pltpu.make_async_remote_copy(
                src_ref=red16_ref,
                dst_ref=out_ref.at[pl.ds(s * ROWS, ROWS), :],
                send_sem=p2_send.at[0],
                recv_sem=p2_recv.at[s],
                device_id=(s,),
                device_id_type=pl.DeviceIdType.MESH,
            )

            @pl.when(s != my_pos)
            def _(recv=recv):
                recv.wait_recv()

        for rdma in p1_rdmas:
            rdma.wait_send()
        for rdma in p2_rdmas:
            rdma.wait_send()

        for j in range(N_DEV):
            @pl.when(j != my_pos)
            def _(j=j):
                pl.semaphore_signal(
                    barrier_sem, inc=1,
                    device_id=(j,), device_id_type=pl.DeviceIdType.MESH,
                )

    return pl.pallas_call(
        body,
        out_shape=jax.ShapeDtypeStruct((m, n), jnp.bfloat16),
        in_specs=[pl.BlockSpec(memory_space=pltpu.VMEM)] * 4,
        out_specs=pl.BlockSpec(memory_space=pltpu.VMEM),
        scratch_shapes=[
            pltpu.VMEM((m, n), jnp.bfloat16),
            pltpu.VMEM((N_DEV, ROWS, n), jnp.bfloat16),
            pltpu.VMEM((ROWS, n), jnp.bfloat16),
            pltpu.SemaphoreType.DMA((N_DEV,)),
            pltpu.SemaphoreType.DMA((N_DEV,)),
            pltpu.SemaphoreType.DMA((N_DEV,)),
            pltpu.SemaphoreType.DMA((N_DEV,)),
        ],
        compiler_params=pltpu.CompilerParams(collective_id=0),
    )(
        x.astype(jnp.bfloat16),
        Wg.astype(jnp.bfloat16),
        Wu.astype(jnp.bfloat16),
        Wd.astype(jnp.bfloat16),
    )


# device time: 18946 ns/iter; 1.0845x vs baseline; 1.0014x over previous
import jax
import jax.numpy as jnp
from jax import lax
from jax.experimental import pallas as pl
from jax.experimental.pallas import tpu as pltpu

N_DEV = 32
ROWS = 8

_SEND_ORDER = [14, 18, 10, 22, 13, 19, 11, 21, 12, 20, 6, 26, 5, 15, 17,
               27, 2, 30, 3, 9, 23, 29, 4, 28, 7, 25, 16, 8, 24, 1, 31]


def kernel(x, Wg, Wu, Wd):
    m, k = x.shape
    _, h_per = Wg.shape
    _, n = Wd.shape

    def body(x_ref, wg_ref, wu_ref, wd_ref, out_ref,
             acc16_ref, comm_ref, red16_ref,
             p1_send, p1_recv, p2_send, p2_recv):
        my_pos = lax.axis_index("i")

        barrier_sem = pltpu.get_barrier_semaphore()
        for j in range(N_DEV):
            @pl.when(j != my_pos)
            def _(j=j):
                pl.semaphore_signal(
                    barrier_sem, inc=1,
                    device_id=(j,), device_id_type=pl.DeviceIdType.MESH,
                )

        gate = jnp.dot(x_ref[...], wg_ref[...],
                       preferred_element_type=jnp.float32)
        up = jnp.dot(x_ref[...], wu_ref[...],
                     preferred_element_type=jnp.float32)
        hidden = (gate * (up * jax.nn.sigmoid(up))).astype(jnp.bfloat16)
        acc16_ref[...] = jnp.dot(
            hidden, wd_ref[...], preferred_element_type=jnp.float32,
        ).astype(jnp.bfloat16)

        pl.semaphore_wait(barrier_sem, N_DEV - 1)

        p1_rdmas = []
        for off in _SEND_ORDER:
            j = (my_pos + off) % N_DEV
            rdma = pltpu.make_async_remote_copy(
                src_ref=acc16_ref.at[pl.ds(j * ROWS, ROWS), :],
                dst_ref=comm_ref.at[my_pos],
                send_sem=p1_send.at[off],
                recv_sem=p1_recv.at[my_pos],
                device_id=(j,),
                device_id_type=pl.DeviceIdType.MESH,
            )
            rdma.start()
            p1_rdmas.append(rdma)

        comm_ref[my_pos] = acc16_ref[pl.ds(my_pos * ROWS, ROWS), :]

        for s in range(N_DEV):
            recv = pltpu.make_async_remote_copy(
                src_ref=comm_ref.at[s],
                dst_ref=comm_ref.at[s],
                send_sem=p1_send.at[0],
                recv_sem=p1_recv.at[s],
                device_id=(s,),
                device_id_type=pl.DeviceIdType.MESH,
            )

            @pl.when(s != my_pos)
            def _(recv=recv):
                recv.wait_recv()

        red = jnp.sum(comm_ref[...].astype(jnp.float32), axis=0)
        red16_ref[...] = red.astype(jnp.bfloat16)

        out_ref[pl.ds(my_pos * ROWS, ROWS), :] = red16_ref[...]
        p2_rdmas = []
        for off in _SEND_ORDER:
            j = (my_pos + off) % N_DEV
            rdma = pltpu.make_async_remote_copy(
                src_ref=red16_ref,
                dst_ref=out_ref.at[pl.ds(my_pos * ROWS, ROWS), :],
                send_sem=p2_send.at[off],
                recv_sem=p2_recv.at[my_pos],
                device_id=(j,),
                device_id_type=pl.DeviceIdType.MESH,
            )
            rdma.start()
            p2_rdmas.append(rdma)

        for s in range(N_DEV):
            recv = pltpu.make_async_remote_copy(
                src_ref=red16_ref,
                dst_ref=out_ref.at[pl.ds(s * ROWS, ROWS), :],
                send_sem=p2_send.at[0],
                recv_sem=p2_recv.at[s],
                device_id=(s,),
                device_id_type=pl.DeviceIdType.MESH,
            )

            @pl.when(s != my_pos)
            def _(recv=recv):
                recv.wait_recv()

        for rdma in p1_rdmas:
            rdma.wait_send()
        for rdma in p2_rdmas:
            rdma.wait_send()

        for j in range(N_DEV):
            @pl.when(j != my_pos)
            def _(j=j):
                pl.semaphore_signal(
                    barrier_sem, inc=1,
                    device_id=(j,), device_id_type=pl.DeviceIdType.MESH,
                )

    return pl.pallas_call(
        body,
        out_shape=jax.ShapeDtypeStruct((m, n), jnp.bfloat16),
        in_specs=[pl.BlockSpec(memory_space=pltpu.VMEM)] * 4,
        out_specs=pl.BlockSpec(memory_space=pltpu.VMEM),
        scratch_shapes=[
            pltpu.VMEM((m, n), jnp.bfloat16),
            pltpu.VMEM((N_DEV, ROWS, n), jnp.bfloat16),
            pltpu.VMEM((ROWS, n), jnp.bfloat16),
            pltpu.SemaphoreType.DMA((N_DEV,)),
            pltpu.SemaphoreType.DMA((N_DEV,)),
            pltpu.SemaphoreType.DMA((N_DEV,)),
            pltpu.SemaphoreType.DMA((N_DEV,)),
        ],
        compiler_params=pltpu.CompilerParams(collective_id=0),
    )(
        x.astype(jnp.bfloat16),
        Wg.astype(jnp.bfloat16),
        Wu.astype(jnp.bfloat16),
        Wd.astype(jnp.bfloat16),
    )
